# Initial kernel scaffold; baseline (speedup 1.0000x reference)
import jax
import jax.numpy as jnp
from jax.experimental import pallas as pl


def kernel(inputs, embeddings):
    raise NotImplementedError("write your pallas kernel here")


# speedup vs baseline: 6.5174x; 6.5174x over previous
import functools

import jax
import jax.numpy as jnp
from jax import lax
from jax.experimental import pallas as pl
from jax.experimental.pallas import tpu as pltpu
from jax.experimental.pallas import tpu_sc as plsc

D = 32
K = 8192
N = 16384
R = 128
G = N // R

_NC = 2
_NS = 16
_NW = _NC * _NS
_CHUNK = 128
_ROWS_PER_W = (N // _CHUNK) // _NW


def _distance_body(x_ref, emb_ref, enc_ref, idx_ref, loss_ref):
    x = x_ref[...]
    emb = emb_ref[...]
    sim = jnp.dot(x, emb, preferred_element_type=jnp.float32)
    x2 = jnp.sum(x ** 2, axis=1, keepdims=True)
    e2 = jnp.sum(emb ** 2, axis=0, keepdims=True)
    dist = x2 - 2.0 * sim + e2
    idx = jnp.argmin(dist, axis=1).astype(jnp.int32)
    ids = lax.broadcasted_iota(jnp.int32, (R, K), 1)
    enc_ref[...] = (ids == idx[:, None]).astype(jnp.float32)
    idx_ref[...] = idx.reshape(1, 1, R)

    s = jnp.sum(jnp.min(dist, axis=1))

    @pl.when(pl.program_id(0) == 0)
    def _():
        loss_ref[0, 0] = 0.0

    loss_ref[0, 0] += s

    @pl.when(pl.program_id(0) == pl.num_programs(0) - 1)
    def _():
        loss_ref[0, 0] *= 1.0 / float(N * D)


def _sc_gather(table, idx2d):
    mesh = plsc.VectorSubcoreMesh(core_axis_name="c", subcore_axis_name="s")

    @functools.partial(
        pl.kernel,
        mesh=mesh,
        out_type=jax.ShapeDtypeStruct((N // _CHUNK, _CHUNK, D), jnp.float32),
        scratch_types=[
            pltpu.VMEM((_ROWS_PER_W, _CHUNK), jnp.int32),
            pltpu.VMEM((_ROWS_PER_W, _CHUNK, D), jnp.float32),
            pltpu.SemaphoreType.DMA,
        ],
        compiler_params=pltpu.CompilerParams(use_tc_tiling_on_sc=False),
    )
    def gather_kernel(table_hbm, idx_hbm, out_hbm, idx_v, rows_v, sem):
        wid = lax.axis_index("s") * _NC + lax.axis_index("c")
        base = wid * _ROWS_PER_W
        pltpu.sync_copy(idx_hbm.at[pl.ds(base, _ROWS_PER_W)], idx_v)
        copies = [
            pltpu.async_copy(table_hbm.at[idx_v.at[j]], rows_v.at[j], sem)
            for j in range(_ROWS_PER_W)
        ]
        for c in copies:
            c.wait()
        pltpu.sync_copy(rows_v, out_hbm.at[pl.ds(base, _ROWS_PER_W)])

    return gather_kernel(table, idx2d)


def kernel(inputs, embeddings):
    flat = inputs.reshape(N, D)

    encodings, idx3, loss_out = pl.pallas_call(
        _distance_body,
        grid=(G,),
        in_specs=[
            pl.BlockSpec((R, D), lambda i: (i, 0)),
            pl.BlockSpec((D, K), lambda i: (0, 0)),
        ],
        out_specs=[
            pl.BlockSpec((R, K), lambda i: (i, 0)),
            pl.BlockSpec((1, 1, R), lambda i: (i, 0, 0)),
            pl.BlockSpec(memory_space=pltpu.SMEM),
        ],
        out_shape=[
            jax.ShapeDtypeStruct((N, K), jnp.float32),
            jax.ShapeDtypeStruct((G, 1, R), jnp.int32),
            jax.ShapeDtypeStruct((1, 1), jnp.float32),
        ],
    )(flat, embeddings)

    idx2d = idx3.reshape(N // _CHUNK, _CHUNK)
    quantized = _sc_gather(embeddings.T, idx2d).reshape(inputs.shape)
    encoding_indices = idx3.reshape(inputs.shape[:-1])
    loss = loss_out[0, 0]
    return quantized, encodings, encoding_indices, loss


# speedup vs baseline: 6.8951x; 1.0580x over previous
import functools

import jax
import jax.numpy as jnp
from jax import lax
from jax.experimental import pallas as pl
from jax.experimental.pallas import tpu as pltpu
from jax.experimental.pallas import tpu_sc as plsc

D = 32
K = 8192
N = 16384
R = 128
G = N // R

_NC = 2
_NS = 16
_NW = _NC * _NS
_CHUNK = 128
_ROWS_PER_W = (N // _CHUNK) // _NW


def _distance_body(x_ref, emb_ref, enc_ref, idx_ref, loss_ref, e2_ref):
    @pl.when(pl.program_id(0) == 0)
    def _():
        emb0 = emb_ref[...]
        e2_ref[...] = jnp.sum(emb0 ** 2, axis=0, keepdims=True)

    y = x_ref[...] * -2.0
    emb = emb_ref[...]
    simy = jnp.dot(y, emb, preferred_element_type=jnp.float32)
    x2 = jnp.sum(y * y, axis=1, keepdims=True) * 0.25
    e2 = e2_ref[...]
    dist = x2 + simy + e2
    idx = jnp.argmin(dist, axis=1).astype(jnp.int32)
    ids = lax.broadcasted_iota(jnp.int32, (R, K), 1)
    enc_ref[...] = (ids == idx[:, None]).astype(jnp.float32)
    idx_ref[...] = idx.reshape(1, 1, R)

    s = jnp.sum(jnp.min(dist, axis=1))

    @pl.when(pl.program_id(0) == 0)
    def _():
        loss_ref[0, 0] = 0.0

    loss_ref[0, 0] += s

    @pl.when(pl.program_id(0) == pl.num_programs(0) - 1)
    def _():
        loss_ref[0, 0] *= 1.0 / float(N * D)


def _sc_gather(table, idx2d):
    mesh = plsc.VectorSubcoreMesh(core_axis_name="c", subcore_axis_name="s")

    @functools.partial(
        pl.kernel,
        mesh=mesh,
        out_type=jax.ShapeDtypeStruct((N // _CHUNK, _CHUNK, D), jnp.float32),
        scratch_types=[
            pltpu.VMEM((_ROWS_PER_W, _CHUNK), jnp.int32),
            pltpu.VMEM((_ROWS_PER_W, _CHUNK, D), jnp.float32),
            pltpu.SemaphoreType.DMA,
        ],
        compiler_params=pltpu.CompilerParams(use_tc_tiling_on_sc=False),
    )
    def gather_kernel(table_hbm, idx_hbm, out_hbm, idx_v, rows_v, sem):
        wid = lax.axis_index("s") * _NC + lax.axis_index("c")
        base = wid * _ROWS_PER_W
        pltpu.sync_copy(idx_hbm.at[pl.ds(base, _ROWS_PER_W)], idx_v)
        copies = [
            pltpu.async_copy(table_hbm.at[idx_v.at[j]], rows_v.at[j], sem)
            for j in range(_ROWS_PER_W)
        ]
        for c in copies:
            c.wait()
        pltpu.sync_copy(rows_v, out_hbm.at[pl.ds(base, _ROWS_PER_W)])

    return gather_kernel(table, idx2d)


def kernel(inputs, embeddings):
    flat = inputs.reshape(N, D)

    encodings, idx3, loss_out = pl.pallas_call(
        _distance_body,
        grid=(G,),
        in_specs=[
            pl.BlockSpec((R, D), lambda i: (i, 0)),
            pl.BlockSpec((D, K), lambda i: (0, 0)),
        ],
        out_specs=[
            pl.BlockSpec((R, K), lambda i: (i, 0)),
            pl.BlockSpec((1, 1, R), lambda i: (i, 0, 0)),
            pl.BlockSpec(memory_space=pltpu.SMEM),
        ],
        out_shape=[
            jax.ShapeDtypeStruct((N, K), jnp.float32),
            jax.ShapeDtypeStruct((G, 1, R), jnp.int32),
            jax.ShapeDtypeStruct((1, 1), jnp.float32),
        ],
        scratch_shapes=[pltpu.VMEM((1, K), jnp.float32)],
    )(flat, embeddings)

    idx2d = idx3.reshape(N // _CHUNK, _CHUNK)
    quantized = _sc_gather(embeddings.T, idx2d).reshape(inputs.shape)
    encoding_indices = idx3.reshape(inputs.shape[:-1])
    loss = loss_out[0, 0]
    return quantized, encodings, encoding_indices, loss


# speedup vs baseline: 7.4751x; 1.0841x over previous
import functools

import jax
import jax.numpy as jnp
from jax import lax
from jax.experimental import pallas as pl
from jax.experimental.pallas import tpu as pltpu
from jax.experimental.pallas import tpu_sc as plsc

D = 32
K = 8192
N = 16384
R = 256
G = N // R

_NC = 2
_NS = 16
_NW = _NC * _NS
_CHUNK = 128
_ROWS_PER_W = (N // _CHUNK) // _NW


def _distance_body(x_ref, emb_ref, enc_ref, idx_ref, loss_ref, e2_ref):
    @pl.when(pl.program_id(0) == 0)
    def _():
        emb0 = emb_ref[...]
        e2_ref[...] = jnp.sum(emb0 ** 2, axis=0, keepdims=True)

    y = x_ref[...] * -2.0
    emb = emb_ref[...]
    simy = jnp.dot(y, emb, preferred_element_type=jnp.float32)
    x2 = jnp.sum(y * y, axis=1, keepdims=True) * 0.25
    e2 = e2_ref[...]
    dist = x2 + simy + e2
    idx = jnp.argmin(dist, axis=1).astype(jnp.int32)
    ids = lax.broadcasted_iota(jnp.int32, (R, K), 1)
    enc_ref[...] = (ids == idx[:, None]).astype(jnp.float32)
    idx_ref[...] = idx.reshape(1, 1, R)

    s = jnp.sum(jnp.min(dist, axis=1))

    @pl.when(pl.program_id(0) == 0)
    def _():
        loss_ref[0, 0] = 0.0

    loss_ref[0, 0] += s

    @pl.when(pl.program_id(0) == pl.num_programs(0) - 1)
    def _():
        loss_ref[0, 0] *= 1.0 / float(N * D)


def _sc_gather(table, idx2d):
    mesh = plsc.VectorSubcoreMesh(core_axis_name="c", subcore_axis_name="s")

    @functools.partial(
        pl.kernel,
        mesh=mesh,
        out_type=jax.ShapeDtypeStruct((N // _CHUNK, _CHUNK, D), jnp.float32),
        scratch_types=[
            pltpu.VMEM((_ROWS_PER_W, _CHUNK), jnp.int32),
            pltpu.VMEM((_ROWS_PER_W, _CHUNK, D), jnp.float32),
            pltpu.SemaphoreType.DMA,
        ],
        compiler_params=pltpu.CompilerParams(use_tc_tiling_on_sc=False),
    )
    def gather_kernel(table_hbm, idx_hbm, out_hbm, idx_v, rows_v, sem):
        wid = lax.axis_index("s") * _NC + lax.axis_index("c")
        base = wid * _ROWS_PER_W
        pltpu.sync_copy(idx_hbm.at[pl.ds(base, _ROWS_PER_W)], idx_v)
        copies = [
            pltpu.async_copy(table_hbm.at[idx_v.at[j]], rows_v.at[j], sem)
            for j in range(_ROWS_PER_W)
        ]
        for c in copies:
            c.wait()
        pltpu.sync_copy(rows_v, out_hbm.at[pl.ds(base, _ROWS_PER_W)])

    return gather_kernel(table, idx2d)


def kernel(inputs, embeddings):
    flat = inputs.reshape(N, D)

    encodings, idx3, loss_out = pl.pallas_call(
        _distance_body,
        grid=(G,),
        in_specs=[
            pl.BlockSpec((R, D), lambda i: (i, 0)),
            pl.BlockSpec((D, K), lambda i: (0, 0)),
        ],
        out_specs=[
            pl.BlockSpec((R, K), lambda i: (i, 0)),
            pl.BlockSpec((1, 1, R), lambda i: (i, 0, 0)),
            pl.BlockSpec(memory_space=pltpu.SMEM),
        ],
        out_shape=[
            jax.ShapeDtypeStruct((N, K), jnp.float32),
            jax.ShapeDtypeStruct((G, 1, R), jnp.int32),
            jax.ShapeDtypeStruct((1, 1), jnp.float32),
        ],
        scratch_shapes=[pltpu.VMEM((1, K), jnp.float32)],
        compiler_params=pltpu.CompilerParams(
            vmem_limit_bytes=100 * 1024 * 1024,
        ),
    )(flat, embeddings)

    idx2d = idx3.reshape(N // _CHUNK, _CHUNK)
    quantized = _sc_gather(embeddings.T, idx2d).reshape(inputs.shape)
    encoding_indices = idx3.reshape(inputs.shape[:-1])
    loss = loss_out[0, 0]
    return quantized, encodings, encoding_indices, loss


# speedup vs baseline: 7.7004x; 1.0301x over previous
import functools

import jax
import jax.numpy as jnp
from jax import lax
from jax.experimental import pallas as pl
from jax.experimental.pallas import tpu as pltpu
from jax.experimental.pallas import tpu_sc as plsc

D = 32
K = 8192
N = 16384
R = 512
G = N // R

_NC = 2
_NS = 16
_NW = _NC * _NS
_CHUNK = 128
_ROWS_PER_W = (N // _CHUNK) // _NW


def _distance_body(x_ref, emb_ref, enc_ref, idx_ref, loss_ref, e2_ref):
    @pl.when(pl.program_id(0) == 0)
    def _():
        emb0 = emb_ref[...]
        e2_ref[...] = jnp.sum(emb0 ** 2, axis=0, keepdims=True)

    y = x_ref[...] * -2.0
    emb = emb_ref[...]
    simy = jnp.dot(y, emb, preferred_element_type=jnp.float32)
    x2 = jnp.sum(y * y, axis=1, keepdims=True) * 0.25
    e2 = e2_ref[...]
    dist = x2 + simy + e2
    idx = jnp.argmin(dist, axis=1).astype(jnp.int32)
    ids = lax.broadcasted_iota(jnp.int32, (R, K), 1)
    enc_ref[...] = (ids == idx[:, None]).astype(jnp.float32)
    idx_ref[...] = idx.reshape(1, 1, R)

    s = jnp.sum(jnp.min(dist, axis=1))

    @pl.when(pl.program_id(0) == 0)
    def _():
        loss_ref[0, 0] = 0.0

    loss_ref[0, 0] += s

    @pl.when(pl.program_id(0) == pl.num_programs(0) - 1)
    def _():
        loss_ref[0, 0] *= 1.0 / float(N * D)


def _sc_gather(table, idx2d):
    mesh = plsc.VectorSubcoreMesh(core_axis_name="c", subcore_axis_name="s")

    @functools.partial(
        pl.kernel,
        mesh=mesh,
        out_type=jax.ShapeDtypeStruct((N // _CHUNK, _CHUNK, D), jnp.float32),
        scratch_types=[
            pltpu.VMEM((_ROWS_PER_W, _CHUNK), jnp.int32),
            pltpu.VMEM((_ROWS_PER_W, _CHUNK, D), jnp.float32),
            pltpu.SemaphoreType.DMA,
        ],
        compiler_params=pltpu.CompilerParams(use_tc_tiling_on_sc=False),
    )
    def gather_kernel(table_hbm, idx_hbm, out_hbm, idx_v, rows_v, sem):
        wid = lax.axis_index("s") * _NC + lax.axis_index("c")
        base = wid * _ROWS_PER_W
        pltpu.sync_copy(idx_hbm.at[pl.ds(base, _ROWS_PER_W)], idx_v)
        copies = [
            pltpu.async_copy(table_hbm.at[idx_v.at[j]], rows_v.at[j], sem)
            for j in range(_ROWS_PER_W)
        ]
        for c in copies:
            c.wait()
        pltpu.sync_copy(rows_v, out_hbm.at[pl.ds(base, _ROWS_PER_W)])

    return gather_kernel(table, idx2d)


def kernel(inputs, embeddings):
    flat = inputs.reshape(N, D)

    encodings, idx3, loss_out = pl.pallas_call(
        _distance_body,
        grid=(G,),
        in_specs=[
            pl.BlockSpec((R, D), lambda i: (i, 0)),
            pl.BlockSpec((D, K), lambda i: (0, 0)),
        ],
        out_specs=[
            pl.BlockSpec((R, K), lambda i: (i, 0)),
            pl.BlockSpec((1, 1, R), lambda i: (i, 0, 0)),
            pl.BlockSpec(memory_space=pltpu.SMEM),
        ],
        out_shape=[
            jax.ShapeDtypeStruct((N, K), jnp.float32),
            jax.ShapeDtypeStruct((G, 1, R), jnp.int32),
            jax.ShapeDtypeStruct((1, 1), jnp.float32),
        ],
        scratch_shapes=[pltpu.VMEM((1, K), jnp.float32)],
        compiler_params=pltpu.CompilerParams(
            vmem_limit_bytes=100 * 1024 * 1024,
        ),
    )(flat, embeddings)

    idx2d = idx3.reshape(N // _CHUNK, _CHUNK)
    quantized = _sc_gather(embeddings.T, idx2d).reshape(inputs.shape)
    encoding_indices = idx3.reshape(inputs.shape[:-1])
    loss = loss_out[0, 0]
    return quantized, encodings, encoding_indices, loss
